# baseline (speedup 1.0000x reference)
import functools

import jax
import jax.numpy as jnp
from jax import lax
from jax.experimental import pallas as pl
from jax.experimental.pallas import tpu as pltpu
from jax.experimental.pallas import tpu_sc as plsc

N = 10000
D = 128
E = 320000

NC = 2
NS = 16
NW = NC * NS
EPW = E // NW
EB = 125
NBLK = EPW // EB
RPS = N // NS
LANES = 16

_mesh = plsc.VectorSubcoreMesh(core_axis_name="c", subcore_axis_name="s")

_cp = pltpu.CompilerParams()
if "needs_layout_passes" in pltpu.CompilerParams.__dataclass_fields__:
    import dataclasses as _dc
    _cp = _dc.replace(_cp, needs_layout_passes=False)


@functools.partial(
    pl.kernel,
    out_type=jax.ShapeDtypeStruct((NC, N, D), jnp.float32),
    mesh=_mesh,
    compiler_params=_cp,
    scratch_types=[
        pltpu.VMEM((NBLK, EB), jnp.int32),
        pltpu.VMEM((NBLK, EB), jnp.int32),
        pltpu.VMEM((NBLK, EB), jnp.float32),
        pltpu.VMEM((EB, D), jnp.float32),
        pltpu.VMEM_SHARED((N, D), jnp.float32),
    ],
)
def _agg(g_hbm, src_hbm, dst_hbm, w_hbm, out_hbm,
         src_v, dst_v, w_v, rows_v, acc_sh):
    cid = lax.axis_index("c")
    sid = lax.axis_index("s")
    wid = sid * NC + cid

    pltpu.sync_copy(src_hbm.at[wid], src_v)
    pltpu.sync_copy(dst_hbm.at[wid], dst_v)
    pltpu.sync_copy(w_hbm.at[wid], w_v)

    @pl.loop(0, EB)
    def _(r):
        for j in range(D // LANES):
            rows_v[r, pl.ds(j * LANES, LANES)] = jnp.zeros((LANES,), jnp.float32)

    @pl.loop(0, RPS // EB)
    def _(k):
        pltpu.sync_copy(rows_v, acc_sh.at[pl.ds(sid * RPS + k * EB, EB)])

    plsc.subcore_barrier()

    @pl.loop(0, NBLK)
    def _(b):
        pltpu.sync_copy(g_hbm.at[src_v.at[b]], rows_v)

        @pl.loop(0, EB, step=5)
        def _(e0):
            wv = [plsc.load_gather(w_v.at[b],
                                   [jnp.full((LANES,), e0 + u, jnp.int32)])
                  for u in range(5)]
            for u in range(5):
                for j in range(D // LANES):
                    sl = pl.ds(j * LANES, LANES)
                    rows_v[e0 + u, sl] = rows_v[e0 + u, sl] * wv[u]

        pltpu.sync_copy(rows_v, acc_sh.at[dst_v.at[b]], add=True)

    plsc.subcore_barrier()

    pltpu.sync_copy(acc_sh.at[pl.ds(sid * 624, 624)],
                    out_hbm.at[cid, pl.ds(sid * 624, 624)])

    @pl.when(sid == NS - 1)
    def _():
        pltpu.sync_copy(acc_sh.at[pl.ds(NS * 624, N - NS * 624)],
                        out_hbm.at[cid, pl.ds(NS * 624, N - NS * 624)])


_BLK = 1000


def _mm_bias_body(x_ref, w_ref, b_ref, o_ref):
    o_ref[...] = (jnp.dot(x_ref[...], w_ref[...],
                          preferred_element_type=jnp.float32) + b_ref[...])


def _comb_mm_body(p_ref, w_ref, b_ref, o_ref):
    h = jnp.maximum(p_ref[0] + p_ref[1], 0.0)
    o_ref[...] = (jnp.dot(h, w_ref[...],
                          preferred_element_type=jnp.float32) + b_ref[...])


def _add_body(p_ref, o_ref):
    o_ref[...] = p_ref[0] + p_ref[1]


def _mm_bias(x, W, b):
    return pl.pallas_call(
        _mm_bias_body,
        grid=(N // _BLK,),
        in_specs=[pl.BlockSpec((_BLK, D), lambda i: (i, 0)),
                  pl.BlockSpec((D, D), lambda i: (0, 0)),
                  pl.BlockSpec((1, D), lambda i: (0, 0))],
        out_specs=pl.BlockSpec((_BLK, D), lambda i: (i, 0)),
        out_shape=jax.ShapeDtypeStruct((N, D), jnp.float32),
    )(x, W, b.reshape(1, D))


def _comb_mm(p, W, b):
    return pl.pallas_call(
        _comb_mm_body,
        grid=(N // _BLK,),
        in_specs=[pl.BlockSpec((NC, _BLK, D), lambda i: (0, i, 0)),
                  pl.BlockSpec((D, D), lambda i: (0, 0)),
                  pl.BlockSpec((1, D), lambda i: (0, 0))],
        out_specs=pl.BlockSpec((_BLK, D), lambda i: (i, 0)),
        out_shape=jax.ShapeDtypeStruct((N, D), jnp.float32),
    )(p, W, b.reshape(1, D))


def _final_add(p):
    return pl.pallas_call(
        _add_body,
        grid=(N // _BLK,),
        in_specs=[pl.BlockSpec((NC, _BLK, D), lambda i: (0, i, 0))],
        out_specs=pl.BlockSpec((_BLK, D), lambda i: (i, 0)),
        out_shape=jax.ShapeDtypeStruct((N, D), jnp.float32),
    )(p)


def kernel(x, edge_index, edge_w, W1, b1, W2, b2, W3, b3):
    src_r = edge_index[0].reshape(NW, NBLK, EB)
    dst_r = edge_index[1].reshape(NW, NBLK, EB)
    w_r = edge_w.reshape(NW, NBLK, EB)

    g1 = _mm_bias(x, W1, b1)
    p1 = _agg(g1, src_r, dst_r, w_r)
    g2 = _comb_mm(p1, W2, b2)
    p2 = _agg(g2, src_r, dst_r, w_r)
    g3 = _comb_mm(p2, W3, b3)
    p3 = _agg(g3, src_r, dst_r, w_r)
    return _final_add(p3)


# speedup vs baseline: 1.0019x; 1.0019x over previous
import functools

import jax
import jax.numpy as jnp
from jax import lax
from jax.experimental import pallas as pl
from jax.experimental.pallas import tpu as pltpu
from jax.experimental.pallas import tpu_sc as plsc

N = 10000
D = 128
E = 320000

NC = 2
NS = 16
NW = NC * NS
EPW = E // NW
EB = 125
NBLK = EPW // EB
RPS = N // NS
LANES = 16

_mesh = plsc.VectorSubcoreMesh(core_axis_name="c", subcore_axis_name="s")

_cp = pltpu.CompilerParams()
if "needs_layout_passes" in pltpu.CompilerParams.__dataclass_fields__:
    import dataclasses as _dc
    _cp = _dc.replace(_cp, needs_layout_passes=False)


@functools.partial(
    pl.kernel,
    out_type=jax.ShapeDtypeStruct((NC, N, D), jnp.float32),
    mesh=_mesh,
    compiler_params=_cp,
    scratch_types=[
        pltpu.VMEM((NBLK, EB), jnp.int32),
        pltpu.VMEM((NBLK, EB), jnp.int32),
        pltpu.VMEM((NBLK, EB), jnp.float32),
        pltpu.VMEM((EB, D), jnp.float32),
        pltpu.VMEM_SHARED((N, D), jnp.float32),
    ],
)
def _agg(g_hbm, src_hbm, dst_hbm, w_hbm, out_hbm,
         src_v, dst_v, w_v, rows_v, acc_sh):
    cid = lax.axis_index("c")
    sid = lax.axis_index("s")
    wid = sid * NC + cid

    pltpu.sync_copy(src_hbm.at[wid], src_v)
    pltpu.sync_copy(dst_hbm.at[wid], dst_v)
    pltpu.sync_copy(w_hbm.at[wid], w_v)

    @pl.loop(0, EB)
    def _(r):
        for j in range(D // LANES):
            rows_v[r, pl.ds(j * LANES, LANES)] = jnp.zeros((LANES,), jnp.float32)

    @pl.loop(0, RPS // EB)
    def _(k):
        pltpu.sync_copy(rows_v, acc_sh.at[pl.ds(sid * RPS + k * EB, EB)])

    _tail = RPS - (RPS // EB) * EB
    if _tail:
        pltpu.sync_copy(rows_v.at[pl.ds(0, _tail)],
                        acc_sh.at[pl.ds(sid * RPS + (RPS // EB) * EB, _tail)])

    plsc.subcore_barrier()

    def _scale(buf, b):
        @pl.loop(0, EB, step=5)
        def _(e0):
            wv = [plsc.load_gather(w_v.at[b],
                                   [jnp.full((LANES,), e0 + u, jnp.int32)])
                  for u in range(5)]
            for u in range(5):
                for j in range(D // LANES):
                    sl = pl.ds(j * LANES, LANES)
                    buf[e0 + u, sl] = buf[e0 + u, sl] * wv[u]

    @pl.loop(0, NBLK)
    def _(b):
        pltpu.sync_copy(g_hbm.at[src_v.at[b]], rows_v)
        _scale(rows_v, b)
        pltpu.sync_copy(rows_v, acc_sh.at[dst_v.at[b]], add=True)

    plsc.subcore_barrier()

    pltpu.sync_copy(acc_sh.at[pl.ds(sid * 624, 624)],
                    out_hbm.at[cid, pl.ds(sid * 624, 624)])

    @pl.when(sid == NS - 1)
    def _():
        pltpu.sync_copy(acc_sh.at[pl.ds(NS * 624, N - NS * 624)],
                        out_hbm.at[cid, pl.ds(NS * 624, N - NS * 624)])


_BLK = 1000


def _mm_bias_body(x_ref, w_ref, b_ref, o_ref):
    o_ref[...] = (jnp.dot(x_ref[...], w_ref[...],
                          preferred_element_type=jnp.float32) + b_ref[...])


def _comb_mm_body(p_ref, w_ref, b_ref, o_ref):
    h = jnp.maximum(p_ref[0] + p_ref[1], 0.0)
    o_ref[...] = (jnp.dot(h, w_ref[...],
                          preferred_element_type=jnp.float32) + b_ref[...])


def _add_body(p_ref, o_ref):
    o_ref[...] = p_ref[0] + p_ref[1]


def _mm_bias(x, W, b):
    return pl.pallas_call(
        _mm_bias_body,
        grid=(N // _BLK,),
        in_specs=[pl.BlockSpec((_BLK, D), lambda i: (i, 0)),
                  pl.BlockSpec((D, D), lambda i: (0, 0)),
                  pl.BlockSpec((1, D), lambda i: (0, 0))],
        out_specs=pl.BlockSpec((_BLK, D), lambda i: (i, 0)),
        out_shape=jax.ShapeDtypeStruct((N, D), jnp.float32),
    )(x, W, b.reshape(1, D))


def _comb_mm(p, W, b):
    return pl.pallas_call(
        _comb_mm_body,
        grid=(N // _BLK,),
        in_specs=[pl.BlockSpec((NC, _BLK, D), lambda i: (0, i, 0)),
                  pl.BlockSpec((D, D), lambda i: (0, 0)),
                  pl.BlockSpec((1, D), lambda i: (0, 0))],
        out_specs=pl.BlockSpec((_BLK, D), lambda i: (i, 0)),
        out_shape=jax.ShapeDtypeStruct((N, D), jnp.float32),
    )(p, W, b.reshape(1, D))


def _final_add(p):
    return pl.pallas_call(
        _add_body,
        grid=(N // _BLK,),
        in_specs=[pl.BlockSpec((NC, _BLK, D), lambda i: (0, i, 0))],
        out_specs=pl.BlockSpec((_BLK, D), lambda i: (i, 0)),
        out_shape=jax.ShapeDtypeStruct((N, D), jnp.float32),
    )(p)


def kernel(x, edge_index, edge_w, W1, b1, W2, b2, W3, b3):
    src_r = edge_index[0].reshape(NW, NBLK, EB)
    dst_r = edge_index[1].reshape(NW, NBLK, EB)
    w_r = edge_w.reshape(NW, NBLK, EB)

    g1 = _mm_bias(x, W1, b1)
    p1 = _agg(g1, src_r, dst_r, w_r)
    g2 = _comb_mm(p1, W2, b2)
    p2 = _agg(g2, src_r, dst_r, w_r)
    g3 = _comb_mm(p2, W3, b3)
    p3 = _agg(g3, src_r, dst_r, w_r)
    return _final_add(p3)


# speedup vs baseline: 1.4840x; 1.4812x over previous
import functools

import jax
import jax.numpy as jnp
from jax import lax
from jax.experimental import pallas as pl
from jax.experimental.pallas import tpu as pltpu
from jax.experimental.pallas import tpu_sc as plsc

N = 10000
D = 128
E = 320000

NC = 2
NS = 16
NW = NC * NS
EPW = E // NW
EB = 125
NBLK = EPW // EB
NCH = 5
CBLK = NBLK // NCH
RPS = N // NS
LANES = 16

_mesh = plsc.VectorSubcoreMesh(core_axis_name="c", subcore_axis_name="s")

_cp = pltpu.CompilerParams()
if "needs_layout_passes" in pltpu.CompilerParams.__dataclass_fields__:
    import dataclasses as _dc
    _cp = _dc.replace(_cp, needs_layout_passes=False)


@functools.partial(
    pl.kernel,
    out_type=jax.ShapeDtypeStruct((NC, N, D), jnp.float32),
    mesh=_mesh,
    compiler_params=_cp,
    scratch_types=[
        pltpu.VMEM((CBLK, EB), jnp.int32),
        pltpu.VMEM((CBLK, EB), jnp.int32),
        pltpu.VMEM((CBLK, EB), jnp.float32),
        pltpu.VMEM((EB, D), jnp.float32),
        pltpu.VMEM((EB, D), jnp.float32),
        pltpu.VMEM_SHARED((N, D), jnp.float32),
        pltpu.SemaphoreType.DMA,
        pltpu.SemaphoreType.DMA,
    ],
)
def _agg(g_hbm, src_hbm, dst_hbm, w_hbm, out_hbm,
         src_v, dst_v, w_v, rows0_v, rows1_v, acc_sh, sem0, sem1):
    cid = lax.axis_index("c")
    sid = lax.axis_index("s")
    wid = sid * NC + cid

    @pl.loop(0, EB)
    def _(r):
        for j in range(D // LANES):
            rows0_v[r, pl.ds(j * LANES, LANES)] = jnp.zeros((LANES,), jnp.float32)

    @pl.loop(0, RPS // EB)
    def _(k):
        pltpu.sync_copy(rows0_v, acc_sh.at[pl.ds(sid * RPS + k * EB, EB)])

    _tail = RPS - (RPS // EB) * EB
    if _tail:
        pltpu.sync_copy(rows0_v.at[pl.ds(0, _tail)],
                        acc_sh.at[pl.ds(sid * RPS + (RPS // EB) * EB, _tail)])

    plsc.subcore_barrier()

    def _scale(buf, b):
        @pl.loop(0, EB, step=5)
        def _(e0):
            wv = [plsc.load_gather(w_v.at[b],
                                   [jnp.full((LANES,), e0 + u, jnp.int32)])
                  for u in range(5)]
            for u in range(5):
                for j in range(D // LANES):
                    sl = pl.ds(j * LANES, LANES)
                    buf[e0 + u, sl] = buf[e0 + u, sl] * wv[u]

    @pl.loop(0, NCH)
    def _(c):
        pltpu.sync_copy(src_hbm.at[wid, c], src_v)
        pltpu.sync_copy(dst_hbm.at[wid, c], dst_v)
        pltpu.sync_copy(w_hbm.at[wid, c], w_v)

        pltpu.async_copy(g_hbm.at[src_v.at[0]], rows0_v, sem0)

        @pl.loop(0, CBLK, step=2)
        def _(b):
            pltpu.make_async_copy(g_hbm.at[src_v.at[b]], rows0_v, sem0).wait()
            pltpu.async_copy(g_hbm.at[src_v.at[b + 1]], rows1_v, sem1)
            _scale(rows0_v, b)
            pltpu.sync_copy(rows0_v, acc_sh.at[dst_v.at[b]], add=True)

            pltpu.make_async_copy(g_hbm.at[src_v.at[b + 1]], rows1_v, sem1).wait()

            @pl.when(b + 2 < CBLK)
            def _():
                pltpu.async_copy(g_hbm.at[src_v.at[b + 2]], rows0_v, sem0)

            _scale(rows1_v, b + 1)
            pltpu.sync_copy(rows1_v, acc_sh.at[dst_v.at[b + 1]], add=True)

    plsc.subcore_barrier()

    pltpu.sync_copy(acc_sh.at[pl.ds(sid * 624, 624)],
                    out_hbm.at[cid, pl.ds(sid * 624, 624)])

    @pl.when(sid == NS - 1)
    def _():
        pltpu.sync_copy(acc_sh.at[pl.ds(NS * 624, N - NS * 624)],
                        out_hbm.at[cid, pl.ds(NS * 624, N - NS * 624)])


_BLK = 1000


def _mm_bias_body(x_ref, w_ref, b_ref, o_ref):
    o_ref[...] = (jnp.dot(x_ref[...], w_ref[...],
                          preferred_element_type=jnp.float32) + b_ref[...])


def _comb_mm_body(p_ref, w_ref, b_ref, o_ref):
    h = jnp.maximum(p_ref[0] + p_ref[1], 0.0)
    o_ref[...] = (jnp.dot(h, w_ref[...],
                          preferred_element_type=jnp.float32) + b_ref[...])


def _add_body(p_ref, o_ref):
    o_ref[...] = p_ref[0] + p_ref[1]


def _mm_bias(x, W, b):
    return pl.pallas_call(
        _mm_bias_body,
        grid=(N // _BLK,),
        in_specs=[pl.BlockSpec((_BLK, D), lambda i: (i, 0)),
                  pl.BlockSpec((D, D), lambda i: (0, 0)),
                  pl.BlockSpec((1, D), lambda i: (0, 0))],
        out_specs=pl.BlockSpec((_BLK, D), lambda i: (i, 0)),
        out_shape=jax.ShapeDtypeStruct((N, D), jnp.float32),
    )(x, W, b.reshape(1, D))


def _comb_mm(p, W, b):
    return pl.pallas_call(
        _comb_mm_body,
        grid=(N // _BLK,),
        in_specs=[pl.BlockSpec((NC, _BLK, D), lambda i: (0, i, 0)),
                  pl.BlockSpec((D, D), lambda i: (0, 0)),
                  pl.BlockSpec((1, D), lambda i: (0, 0))],
        out_specs=pl.BlockSpec((_BLK, D), lambda i: (i, 0)),
        out_shape=jax.ShapeDtypeStruct((N, D), jnp.float32),
    )(p, W, b.reshape(1, D))


def _final_add(p):
    return pl.pallas_call(
        _add_body,
        grid=(N // _BLK,),
        in_specs=[pl.BlockSpec((NC, _BLK, D), lambda i: (0, i, 0))],
        out_specs=pl.BlockSpec((_BLK, D), lambda i: (i, 0)),
        out_shape=jax.ShapeDtypeStruct((N, D), jnp.float32),
    )(p)


def kernel(x, edge_index, edge_w, W1, b1, W2, b2, W3, b3):
    src_r = edge_index[0].reshape(NW, NCH, CBLK, EB)
    dst_r = edge_index[1].reshape(NW, NCH, CBLK, EB)
    w_r = edge_w.reshape(NW, NCH, CBLK, EB)

    g1 = _mm_bias(x, W1, b1)
    p1 = _agg(g1, src_r, dst_r, w_r)
    g2 = _comb_mm(p1, W2, b2)
    p2 = _agg(g2, src_r, dst_r, w_r)
    g3 = _comb_mm(p2, W3, b3)
    p3 = _agg(g3, src_r, dst_r, w_r)
    return _final_add(p3)
